# Initial kernel scaffold; baseline (speedup 1.0000x reference)
import jax
import jax.numpy as jnp
from jax.experimental import pallas as pl


def kernel(hidden_states, gate_weight, gate_proj, up_proj, down_proj, shared_gate, shared_up, shared_down):
    raise NotImplementedError("write your pallas kernel here")


# speedup vs baseline: 2.2040x; 2.2040x over previous
import functools

import jax
import jax.numpy as jnp
from jax import lax
from jax.experimental import pallas as pl
from jax.experimental.pallas import tpu as pltpu
from jax.experimental.pallas import tpu_sc as plsc

E = 64
K = 2
D = 768
F = 256
TB = 128
RB = 128
NW = 32


def _gate_shared_body(x_ref, gw_ref, sg_ref, su_ref, sd_ref, w_ref, i_ref, sh_ref):
    xb = x_ref[...]
    logits = lax.dot_general(xb, gw_ref[...], (((1,), (1,)), ((), ())),
                             preferred_element_type=jnp.float32)
    m = jnp.max(logits, axis=1, keepdims=True)
    ex = jnp.exp(logits - m)
    p = ex / jnp.sum(ex, axis=1, keepdims=True)
    lane = lax.broadcasted_iota(jnp.int32, p.shape, 1)
    w1 = jnp.max(p, axis=1, keepdims=True)
    i1 = jnp.min(jnp.where(p == w1, lane, E), axis=1, keepdims=True)
    p2 = jnp.where(lane == i1, -1.0, p)
    w2 = jnp.max(p2, axis=1, keepdims=True)
    i2 = jnp.min(jnp.where(p2 == w2, lane, E), axis=1, keepdims=True)
    lane_o = lax.broadcasted_iota(jnp.int32, (RB, 128), 1)
    w_ref[...] = jnp.where(lane_o == 0, w1, 0.0) + jnp.where(lane_o == 1, w2, 0.0)
    i_ref[...] = jnp.where(lane_o == 0, i1, 0) + jnp.where(lane_o == 1, i2, 0)
    g = jnp.dot(xb, sg_ref[...], preferred_element_type=jnp.float32)
    u = jnp.dot(xb, su_ref[...], preferred_element_type=jnp.float32)
    h = jax.nn.silu(g) * u
    sh_ref[...] = jnp.dot(h, sd_ref[...], preferred_element_type=jnp.float32)


def _moe_ffn_body(be_ref, xs_ref, gp_ref, up_ref, dp_ref, out_ref):
    del be_ref
    xb = xs_ref[...]
    g = jnp.dot(xb, gp_ref[0], preferred_element_type=jnp.float32)
    u = jnp.dot(xb, up_ref[0], preferred_element_type=jnp.float32)
    h = jax.nn.silu(g) * u
    out_ref[...] = jnp.dot(h, dp_ref[0], preferred_element_type=jnp.float32)


def _combine_body(w_ref, g0_ref, g1_ref, sh_ref, y_ref):
    w = w_ref[...]
    w0 = w[:, 0:1]
    w1 = w[:, 1:2]
    y_ref[...] = w0 * g0_ref[...] + w1 * g1_ref[...] + sh_ref[...]


def _sc_gather(table, idx, n_rows, chunk):
    rpw = n_rows // NW
    n_ch = rpw // chunk
    mesh = plsc.VectorSubcoreMesh(core_axis_name="c", subcore_axis_name="s")

    @functools.partial(
        pl.kernel,
        out_type=jax.ShapeDtypeStruct((n_rows, D), jnp.float32),
        mesh=mesh,
        scratch_types=[
            pltpu.VMEM((chunk,), jnp.int32),
            pltpu.VMEM((chunk, D), jnp.float32),
            pltpu.SemaphoreType.DMA,
        ],
    )
    def gather_kernel(table_hbm, idx_hbm, out_hbm, idx_v, rows_v, sem):
        wid = lax.axis_index("s") * 2 + lax.axis_index("c")
        base = wid * rpw
        for c in range(n_ch):
            off = base + c * chunk
            pltpu.sync_copy(idx_hbm.at[pl.ds(off, chunk)], idx_v)
            pltpu.async_copy(table_hbm.at[idx_v], rows_v, sem).wait()
            pltpu.sync_copy(rows_v, out_hbm.at[pl.ds(off, chunk)])

    return gather_kernel(table, idx)


def _routing_meta(flat_idx, NB, NPAD):
    TK = flat_idx.shape[0]
    sort_idx = jnp.argsort(flat_idx)
    sorted_expert = jnp.take(flat_idx, sort_idx)
    counts = jnp.zeros((E,), jnp.int32).at[flat_idx].add(1)
    blocks_per_e = (counts + TB - 1) // TB
    cum_blocks = jnp.cumsum(blocks_per_e)
    astart = (cum_blocks - blocks_per_e) * TB
    starts = jnp.cumsum(counts) - counts
    pos = jnp.arange(TK, dtype=jnp.int32) - jnp.take(starts, sorted_expert)
    p_arr = jnp.take(astart, sorted_expert) + pos
    tok_of_sorted = (sort_idx // K).astype(jnp.int32)
    src_row = jnp.zeros((NPAD,), jnp.int32).at[p_arr].set(tok_of_sorted)
    block_expert = jnp.clip(
        jnp.searchsorted(cum_blocks, jnp.arange(NB, dtype=jnp.int32), side="right"),
        0, E - 1).astype(jnp.int32)
    p_flat = jnp.zeros((TK,), jnp.int32).at[sort_idx].set(p_arr.astype(jnp.int32))
    idx_all = p_flat.reshape(TK // K, K).T.reshape(-1)
    return src_row, block_expert, idx_all


def kernel(hidden_states, gate_weight, gate_proj, up_proj, down_proj,
           shared_gate, shared_up, shared_down):
    orig_shape = hidden_states.shape
    x = hidden_states.reshape(-1, D)
    T = x.shape[0]
    TK = T * K
    NB = TK // TB + E
    NPAD = NB * TB

    w_out, i_out, shared_out = pl.pallas_call(
        _gate_shared_body,
        grid=(T // RB,),
        in_specs=[
            pl.BlockSpec((RB, D), lambda i: (i, 0)),
            pl.BlockSpec((E, D), lambda i: (0, 0)),
            pl.BlockSpec((D, F), lambda i: (0, 0)),
            pl.BlockSpec((D, F), lambda i: (0, 0)),
            pl.BlockSpec((F, D), lambda i: (0, 0)),
        ],
        out_specs=[
            pl.BlockSpec((RB, 128), lambda i: (i, 0)),
            pl.BlockSpec((RB, 128), lambda i: (i, 0)),
            pl.BlockSpec((RB, D), lambda i: (i, 0)),
        ],
        out_shape=[
            jax.ShapeDtypeStruct((T, 128), jnp.float32),
            jax.ShapeDtypeStruct((T, 128), jnp.int32),
            jax.ShapeDtypeStruct((T, D), jnp.float32),
        ],
    )(x, gate_weight, shared_gate, shared_up, shared_down)

    flat_idx = i_out[:, :K].reshape(-1)
    src_row, block_expert, idx_all = _routing_meta(flat_idx, NB, NPAD)

    xs = _sc_gather(x, src_row, NPAD, 128)

    out_moe = pl.pallas_call(
        _moe_ffn_body,
        grid_spec=pltpu.PrefetchScalarGridSpec(
            num_scalar_prefetch=1,
            grid=(NB,),
            in_specs=[
                pl.BlockSpec((TB, D), lambda i, be: (i, 0)),
                pl.BlockSpec((1, D, F), lambda i, be: (be[i], 0, 0)),
                pl.BlockSpec((1, D, F), lambda i, be: (be[i], 0, 0)),
                pl.BlockSpec((1, F, D), lambda i, be: (be[i], 0, 0)),
            ],
            out_specs=pl.BlockSpec((TB, D), lambda i, be: (i, 0)),
        ),
        out_shape=jax.ShapeDtypeStruct((NPAD, D), jnp.float32),
    )(block_expert, xs, gate_proj, up_proj, down_proj)

    g = _sc_gather(out_moe, idx_all, TK, 128)

    y = pl.pallas_call(
        _combine_body,
        grid=(T // RB,),
        in_specs=[
            pl.BlockSpec((RB, 128), lambda i: (i, 0)),
            pl.BlockSpec((RB, D), lambda i: (i, 0)),
            pl.BlockSpec((RB, D), lambda i: (T // RB + i, 0)),
            pl.BlockSpec((RB, D), lambda i: (i, 0)),
        ],
        out_specs=pl.BlockSpec((RB, D), lambda i: (i, 0)),
        out_shape=jax.ShapeDtypeStruct((T, D), jnp.float32),
    )(w_out, g, g, shared_out)

    return y.reshape(orig_shape)


# speedup vs baseline: 2.3292x; 1.0568x over previous
import functools

import jax
import jax.numpy as jnp
from jax import lax
from jax.experimental import pallas as pl
from jax.experimental.pallas import tpu as pltpu
from jax.experimental.pallas import tpu_sc as plsc

E = 64
K = 2
D = 768
F = 256
TB = 128
RB = 128
NW = 32


def _gate_shared_body(x_ref, gw_ref, sg_ref, su_ref, sd_ref, w_ref, i_ref, sh_ref):
    xb = x_ref[...]
    logits = lax.dot_general(xb, gw_ref[...], (((1,), (1,)), ((), ())),
                             preferred_element_type=jnp.float32)
    m = jnp.max(logits, axis=1, keepdims=True)
    ex = jnp.exp(logits - m)
    p = ex / jnp.sum(ex, axis=1, keepdims=True)
    lane = lax.broadcasted_iota(jnp.int32, p.shape, 1)
    w1 = jnp.max(p, axis=1, keepdims=True)
    i1 = jnp.min(jnp.where(p == w1, lane, E), axis=1, keepdims=True)
    p2 = jnp.where(lane == i1, -1.0, p)
    w2 = jnp.max(p2, axis=1, keepdims=True)
    i2 = jnp.min(jnp.where(p2 == w2, lane, E), axis=1, keepdims=True)
    lane_o = lax.broadcasted_iota(jnp.int32, (RB, 128), 1)
    w_ref[...] = jnp.where(lane_o == 0, w1, 0.0) + jnp.where(lane_o == 1, w2, 0.0)
    i_ref[...] = jnp.where(lane_o == 0, i1, 0) + jnp.where(lane_o == 1, i2, 0)
    g = jnp.dot(xb, sg_ref[...], preferred_element_type=jnp.float32)
    u = jnp.dot(xb, su_ref[...], preferred_element_type=jnp.float32)
    h = jax.nn.silu(g) * u
    sh_ref[...] = jnp.dot(h, sd_ref[...], preferred_element_type=jnp.float32)


def _moe_ffn_body(be_ref, xs_ref, gp_ref, up_ref, dp_ref, out_ref):
    del be_ref
    xb = xs_ref[...]
    g = jnp.dot(xb, gp_ref[0], preferred_element_type=jnp.float32)
    u = jnp.dot(xb, up_ref[0], preferred_element_type=jnp.float32)
    h = jax.nn.silu(g) * u
    out_ref[...] = jnp.dot(h, dp_ref[0], preferred_element_type=jnp.float32)


def _combine_body(w_ref, g0_ref, g1_ref, sh_ref, y_ref):
    w = w_ref[...]
    w0 = w[:, 0:1]
    w1 = w[:, 1:2]
    y_ref[...] = w0 * g0_ref[...] + w1 * g1_ref[...] + sh_ref[...]


def _sc_gather(table, idx, n_rows, chunk):
    rpw = n_rows // NW
    n_ch = rpw // chunk
    mesh = plsc.VectorSubcoreMesh(core_axis_name="c", subcore_axis_name="s")

    @functools.partial(
        pl.kernel,
        out_type=jax.ShapeDtypeStruct((n_rows, D), jnp.float32),
        mesh=mesh,
        scratch_types=[
            pltpu.VMEM((chunk,), jnp.int32),
            pltpu.VMEM((chunk, D), jnp.float32),
            pltpu.SemaphoreType.DMA,
        ],
    )
    def gather_kernel(table_hbm, idx_hbm, out_hbm, idx_v, rows_v, sem):
        wid = lax.axis_index("s") * 2 + lax.axis_index("c")
        base = wid * rpw
        for c in range(n_ch):
            off = base + c * chunk
            pltpu.sync_copy(idx_hbm.at[pl.ds(off, chunk)], idx_v)
            pltpu.async_copy(table_hbm.at[idx_v], rows_v, sem).wait()
            pltpu.sync_copy(rows_v, out_hbm.at[pl.ds(off, chunk)])

    return gather_kernel(table, idx)


def _routing_meta(flat_idx, NB, NPAD):
    TK = flat_idx.shape[0]
    oh = (flat_idx[:, None] == jnp.arange(E, dtype=jnp.int32)[None, :]).astype(jnp.int32)
    counts = jnp.sum(oh, axis=0)
    occ = jnp.sum((jnp.cumsum(oh, axis=0) - oh) * oh, axis=1)
    blocks_per_e = (counts + TB - 1) // TB
    cum_blocks = jnp.cumsum(blocks_per_e)
    astart = (cum_blocks - blocks_per_e) * TB
    p_flat = jnp.take(astart, flat_idx) + occ
    tok = jnp.arange(TK, dtype=jnp.int32) // K
    src_row = jnp.zeros((NPAD,), jnp.int32).at[p_flat].set(tok)
    block_expert = jnp.clip(
        jnp.searchsorted(cum_blocks, jnp.arange(NB, dtype=jnp.int32), side="right"),
        0, E - 1).astype(jnp.int32)
    idx_all = p_flat.reshape(TK // K, K).T.reshape(-1)
    return src_row, block_expert, idx_all


def kernel(hidden_states, gate_weight, gate_proj, up_proj, down_proj,
           shared_gate, shared_up, shared_down):
    orig_shape = hidden_states.shape
    x = hidden_states.reshape(-1, D)
    T = x.shape[0]
    TK = T * K
    NB = TK // TB + E
    NPAD = NB * TB

    w_out, i_out, shared_out = pl.pallas_call(
        _gate_shared_body,
        grid=(T // RB,),
        in_specs=[
            pl.BlockSpec((RB, D), lambda i: (i, 0)),
            pl.BlockSpec((E, D), lambda i: (0, 0)),
            pl.BlockSpec((D, F), lambda i: (0, 0)),
            pl.BlockSpec((D, F), lambda i: (0, 0)),
            pl.BlockSpec((F, D), lambda i: (0, 0)),
        ],
        out_specs=[
            pl.BlockSpec((RB, 128), lambda i: (i, 0)),
            pl.BlockSpec((RB, 128), lambda i: (i, 0)),
            pl.BlockSpec((RB, D), lambda i: (i, 0)),
        ],
        out_shape=[
            jax.ShapeDtypeStruct((T, 128), jnp.float32),
            jax.ShapeDtypeStruct((T, 128), jnp.int32),
            jax.ShapeDtypeStruct((T, D), jnp.float32),
        ],
    )(x, gate_weight, shared_gate, shared_up, shared_down)

    flat_idx = i_out[:, :K].reshape(-1)
    src_row, block_expert, idx_all = _routing_meta(flat_idx, NB, NPAD)

    xs = _sc_gather(x, src_row, NPAD, 128)

    out_moe = pl.pallas_call(
        _moe_ffn_body,
        grid_spec=pltpu.PrefetchScalarGridSpec(
            num_scalar_prefetch=1,
            grid=(NB,),
            in_specs=[
                pl.BlockSpec((TB, D), lambda i, be: (i, 0)),
                pl.BlockSpec((1, D, F), lambda i, be: (be[i], 0, 0)),
                pl.BlockSpec((1, D, F), lambda i, be: (be[i], 0, 0)),
                pl.BlockSpec((1, F, D), lambda i, be: (be[i], 0, 0)),
            ],
            out_specs=pl.BlockSpec((TB, D), lambda i, be: (i, 0)),
        ),
        out_shape=jax.ShapeDtypeStruct((NPAD, D), jnp.float32),
    )(block_expert, xs, gate_proj, up_proj, down_proj)

    g = _sc_gather(out_moe, idx_all, TK, 128)

    y = pl.pallas_call(
        _combine_body,
        grid=(T // RB,),
        in_specs=[
            pl.BlockSpec((RB, 128), lambda i: (i, 0)),
            pl.BlockSpec((RB, D), lambda i: (i, 0)),
            pl.BlockSpec((RB, D), lambda i: (T // RB + i, 0)),
            pl.BlockSpec((RB, D), lambda i: (i, 0)),
        ],
        out_specs=pl.BlockSpec((RB, D), lambda i: (i, 0)),
        out_shape=jax.ShapeDtypeStruct((T, D), jnp.float32),
    )(w_out, g, g, shared_out)

    return y.reshape(orig_shape)


# speedup vs baseline: 5.1017x; 2.1903x over previous
import functools

import jax
import jax.numpy as jnp
from jax import lax
from jax.experimental import pallas as pl
from jax.experimental.pallas import tpu as pltpu
from jax.experimental.pallas import tpu_sc as plsc

E = 64
K = 2
D = 768
F = 256
TB = 128
RB = 128
NW = 32


def _gate_shared_body(x_ref, gw_ref, sg_ref, su_ref, sd_ref, w_ref, i_ref, sh_ref):
    xb = x_ref[...]
    logits = lax.dot_general(xb, gw_ref[...], (((1,), (1,)), ((), ())),
                             preferred_element_type=jnp.float32)
    m = jnp.max(logits, axis=1, keepdims=True)
    ex = jnp.exp(logits - m)
    p = ex / jnp.sum(ex, axis=1, keepdims=True)
    lane = lax.broadcasted_iota(jnp.int32, p.shape, 1)
    w1 = jnp.max(p, axis=1, keepdims=True)
    i1 = jnp.min(jnp.where(p == w1, lane, E), axis=1, keepdims=True)
    p2 = jnp.where(lane == i1, -1.0, p)
    w2 = jnp.max(p2, axis=1, keepdims=True)
    i2 = jnp.min(jnp.where(p2 == w2, lane, E), axis=1, keepdims=True)
    lane_o = lax.broadcasted_iota(jnp.int32, (RB, 128), 1)
    w_ref[...] = jnp.where(lane_o == 0, w1, 0.0) + jnp.where(lane_o == 1, w2, 0.0)
    i_ref[...] = jnp.where(lane_o == 0, i1, 0) + jnp.where(lane_o == 1, i2, 0)
    g = jnp.dot(xb, sg_ref[...], preferred_element_type=jnp.float32)
    u = jnp.dot(xb, su_ref[...], preferred_element_type=jnp.float32)
    h = jax.nn.silu(g) * u
    sh_ref[...] = jnp.dot(h, sd_ref[...], preferred_element_type=jnp.float32)


def _moe_ffn_body(be_ref, xs_ref, gp_ref, up_ref, dp_ref, out_ref):
    del be_ref
    xb = xs_ref[...]
    g = jnp.dot(xb, gp_ref[0], preferred_element_type=jnp.float32)
    u = jnp.dot(xb, up_ref[0], preferred_element_type=jnp.float32)
    h = jax.nn.silu(g) * u
    out_ref[...] = jnp.dot(h, dp_ref[0], preferred_element_type=jnp.float32)


def _combine_body(w_ref, g0_ref, g1_ref, sh_ref, y_ref):
    w = w_ref[...]
    w0 = w[:, 0:1]
    w1 = w[:, 1:2]
    y_ref[...] = w0 * g0_ref[...] + w1 * g1_ref[...] + sh_ref[...]


def _sc_gather(table, idx, n_rows, chunk):
    rpw = n_rows // NW
    n_ch = rpw // chunk
    mesh = plsc.VectorSubcoreMesh(core_axis_name="c", subcore_axis_name="s")

    @functools.partial(
        pl.kernel,
        out_type=jax.ShapeDtypeStruct((n_rows, D), jnp.float32),
        mesh=mesh,
        scratch_types=[
            pltpu.VMEM((chunk,), jnp.int32),
            pltpu.VMEM((chunk, D), jnp.float32),
            pltpu.SemaphoreType.DMA,
        ],
    )
    def gather_kernel(table_hbm, idx_hbm, out_hbm, idx_v, rows_v, sem):
        wid = lax.axis_index("s") * 2 + lax.axis_index("c")
        base = wid * rpw
        for c in range(n_ch):
            off = base + c * chunk
            pltpu.sync_copy(idx_hbm.at[pl.ds(off, chunk)], idx_v)
            pltpu.async_copy(table_hbm.at[idx_v], rows_v, sem).wait()
            pltpu.sync_copy(rows_v, out_hbm.at[pl.ds(off, chunk)])

    return gather_kernel(table, idx)


def _routing_meta(flat_idx, NB, NPAD):
    TK = flat_idx.shape[0]
    oh = (flat_idx[:, None] == jnp.arange(E, dtype=jnp.int32)[None, :]).astype(jnp.int32)
    counts = jnp.sum(oh, axis=0)
    occ = jnp.sum((jnp.cumsum(oh, axis=0) - oh) * oh, axis=1)
    blocks_per_e = (counts + TB - 1) // TB
    cum_blocks = jnp.cumsum(blocks_per_e)
    astart = (cum_blocks - blocks_per_e) * TB
    p_flat = jnp.take(astart, flat_idx) + occ
    tok = jnp.arange(TK, dtype=jnp.int32) // K
    src_row = (jnp.arange(NPAD, dtype=jnp.int32) % (TK // K)).at[p_flat].set(tok)
    block_expert = jnp.clip(
        jnp.searchsorted(cum_blocks, jnp.arange(NB, dtype=jnp.int32), side="right"),
        0, E - 1).astype(jnp.int32)
    idx_all = p_flat.reshape(TK // K, K).T.reshape(-1)
    return src_row, block_expert, idx_all


def kernel(hidden_states, gate_weight, gate_proj, up_proj, down_proj,
           shared_gate, shared_up, shared_down):
    orig_shape = hidden_states.shape
    x = hidden_states.reshape(-1, D)
    T = x.shape[0]
    TK = T * K
    NB = TK // TB + E
    NPAD = NB * TB

    w_out, i_out, shared_out = pl.pallas_call(
        _gate_shared_body,
        grid=(T // RB,),
        in_specs=[
            pl.BlockSpec((RB, D), lambda i: (i, 0)),
            pl.BlockSpec((E, D), lambda i: (0, 0)),
            pl.BlockSpec((D, F), lambda i: (0, 0)),
            pl.BlockSpec((D, F), lambda i: (0, 0)),
            pl.BlockSpec((F, D), lambda i: (0, 0)),
        ],
        out_specs=[
            pl.BlockSpec((RB, 128), lambda i: (i, 0)),
            pl.BlockSpec((RB, 128), lambda i: (i, 0)),
            pl.BlockSpec((RB, D), lambda i: (i, 0)),
        ],
        out_shape=[
            jax.ShapeDtypeStruct((T, 128), jnp.float32),
            jax.ShapeDtypeStruct((T, 128), jnp.int32),
            jax.ShapeDtypeStruct((T, D), jnp.float32),
        ],
    )(x, gate_weight, shared_gate, shared_up, shared_down)

    flat_idx = i_out[:, :K].reshape(-1)
    src_row, block_expert, idx_all = _routing_meta(flat_idx, NB, NPAD)

    xs = _sc_gather(x, src_row, NPAD, 128)

    out_moe = pl.pallas_call(
        _moe_ffn_body,
        grid_spec=pltpu.PrefetchScalarGridSpec(
            num_scalar_prefetch=1,
            grid=(NB,),
            in_specs=[
                pl.BlockSpec((TB, D), lambda i, be: (i, 0)),
                pl.BlockSpec((1, D, F), lambda i, be: (be[i], 0, 0)),
                pl.BlockSpec((1, D, F), lambda i, be: (be[i], 0, 0)),
                pl.BlockSpec((1, F, D), lambda i, be: (be[i], 0, 0)),
            ],
            out_specs=pl.BlockSpec((TB, D), lambda i, be: (i, 0)),
        ),
        out_shape=jax.ShapeDtypeStruct((NPAD, D), jnp.float32),
    )(block_expert, xs, gate_proj, up_proj, down_proj)

    g = _sc_gather(out_moe, idx_all, TK, 128)

    y = pl.pallas_call(
        _combine_body,
        grid=(T // RB,),
        in_specs=[
            pl.BlockSpec((RB, 128), lambda i: (i, 0)),
            pl.BlockSpec((RB, D), lambda i: (i, 0)),
            pl.BlockSpec((RB, D), lambda i: (T // RB + i, 0)),
            pl.BlockSpec((RB, D), lambda i: (i, 0)),
        ],
        out_specs=pl.BlockSpec((RB, D), lambda i: (i, 0)),
        out_shape=jax.ShapeDtypeStruct((T, D), jnp.float32),
    )(w_out, g, g, shared_out)

    return y.reshape(orig_shape)
